# baseline (speedup 1.0000x reference)
import functools

import jax
import jax.numpy as jnp
from jax import lax
from jax.experimental import pallas as pl
from jax.experimental.pallas import tpu as pltpu
from jax.experimental.pallas import tpu_sc as plsc

NC = 2
NS = 16
L = 16


@functools.partial(jax.jit, static_argnames=("n_rows", "c"))
def _pack_bf16(x2d, *, n_rows, c):
    nw = NC * NS
    rpw = n_rows // nw
    ng = c // (2 * L)

    mesh = plsc.VectorSubcoreMesh(
        core_axis_name="c", subcore_axis_name="s", num_cores=NC,
        num_subcores=NS)

    @functools.partial(
        pl.kernel,
        mesh=mesh,
        compiler_params=pltpu.CompilerParams(use_tc_tiling_on_sc=False),
        out_type=jax.ShapeDtypeStruct((n_rows, c // 2), jnp.int32),
        scratch_types=[
            pltpu.VMEM((rpw, c), jnp.float32),
            pltpu.VMEM((rpw, c // 2), jnp.int32),
        ],
    )
    def run(x_hbm, out_hbm, inb, outb):
        wid = lax.axis_index("s") * NC + lax.axis_index("c")
        base = wid * rpw
        pltpu.sync_copy(x_hbm.at[pl.ds(base, rpw)], inb)

        one = jnp.full((L,), 1, jnp.int32)
        half = jnp.full((L,), 0x7FFF, jnp.int32)
        himask = jnp.full((L,), -65536, jnp.int32)

        def rnd(v):
            u = lax.bitcast_convert_type(v, jnp.int32)
            return u + half + (lax.shift_right_logical(u, 16) & one)

        @pl.loop(0, rpw, unroll=4)
        def _rows(rr):
            for g in range(ng):
                ra = rnd(inb[rr, pl.ds(g * L, L)])
                rb = rnd(inb[rr, pl.ds(c // 2 + g * L, L)])
                outb[rr, pl.ds(g * L, L)] = (
                    lax.shift_right_logical(ra, 16) | (rb & himask))

        pltpu.sync_copy(outb, out_hbm.at[pl.ds(base, rpw)])

    return run(x2d)


@functools.partial(jax.jit, static_argnames=("n_rows", "k", "c"))
def _pooled_gather(x_flat, gidx, *, n_rows, k, c):
    nw = NC * NS
    rpw = n_rows // nw
    gch = 128 // k
    nch = rpw // gch
    cv = c // L

    mesh = plsc.VectorSubcoreMesh(
        core_axis_name="c", subcore_axis_name="s", num_cores=NC,
        num_subcores=NS)

    ng = c // (2 * L)

    @functools.partial(
        pl.kernel,
        mesh=mesh,
        compiler_params=pltpu.CompilerParams(use_tc_tiling_on_sc=False),
        out_type=jax.ShapeDtypeStruct((n_rows * c,), jnp.float32),
        scratch_types=[
            pltpu.VMEM((rpw * k // 128, 128), jnp.int32),
            pltpu.VMEM((gch * k, c // 2), jnp.int32),
            pltpu.VMEM((gch * k, c // 2), jnp.int32),
            pltpu.VMEM((gch * k, c // 2), jnp.int32),
            pltpu.VMEM((gch * k, c // 2), jnp.int32),
            pltpu.VMEM((rpw * c,), jnp.float32),
            pltpu.SemaphoreType.DMA,
            pltpu.SemaphoreType.DMA,
            pltpu.SemaphoreType.DMA,
            pltpu.SemaphoreType.DMA,
        ],
    )
    def run(x_hbm, gidx_hbm, out_hbm, idx_v, gbuf0, gbuf1, gbuf2, gbuf3,
            obuf, sem0, sem1, sem2, sem3):
        wid = lax.axis_index("s") * NC + lax.axis_index("c")
        base = wid * rpw
        pltpu.sync_copy(
            gidx_hbm.at[pl.ds(wid * (rpw * k // 128), rpw * k // 128)], idx_v)

        himask = jnp.full((L,), -65536, jnp.int32)

        def issue(i, gbuf, sem):
            pltpu.async_copy(x_hbm.at[idx_v.at[i]], gbuf, sem)

        def drain(gbuf, sem):
            pltpu.make_async_copy(
                x_hbm.at[pl.ds(0, gch * k)], gbuf, sem).wait()

        def accumulate(i, gbuf):
            def acc_body(j, carry):
                new = list(carry)
                for r in range(gch):
                    for g in range(ng):
                        w = gbuf[r * k + j, pl.ds(g * L, L)]
                        ev = lax.bitcast_convert_type(
                            lax.shift_left(w, 16), jnp.float32)
                        od = lax.bitcast_convert_type(w & himask, jnp.float32)
                        new[(r * ng + g) * 2] += ev
                        new[(r * ng + g) * 2 + 1] += od
                return tuple(new)
            acc = lax.fori_loop(
                0, k, acc_body,
                tuple(jnp.zeros((L,), jnp.float32) for _ in range(gch * cv)),
                unroll=2)
            for r in range(gch):
                row_off = (i * gch + r) * c
                for g in range(ng):
                    obuf[pl.ds(row_off + g * L, L)] = (
                        acc[(r * ng + g) * 2] * (1.0 / k))
                    obuf[pl.ds(row_off + c // 2 + g * L, L)] = (
                        acc[(r * ng + g) * 2 + 1] * (1.0 / k))

        bufs = (gbuf0, gbuf1, gbuf2, gbuf3)
        sems = (sem0, sem1, sem2, sem3)
        nbuf = 4
        for b in range(nbuf - 1):
            issue(b, bufs[b], sems[b])

        @pl.loop(0, nch, step=nbuf)
        def _chunks(i):
            for b in range(nbuf):
                nxt = i + b + nbuf - 1

                @pl.when(nxt < nch)
                def _(b=b, nxt=nxt):
                    issue(nxt, bufs[(b + nbuf - 1) % nbuf],
                          sems[(b + nbuf - 1) % nbuf])

                drain(bufs[b], sems[b])
                accumulate(i + b, bufs[b])

        pltpu.sync_copy(obuf, out_hbm.at[pl.ds(base * c, rpw * c)])

    return run(x_flat, gidx)


def kernel(x, idx):
    b, n, c = x.shape
    k = idx.shape[-1]
    x_flat = _pack_bf16(x.reshape(b * n, c), n_rows=b * n, c=c)
    offs = (jnp.arange(b, dtype=jnp.int32) * n)[:, None, None]
    gidx = (idx.astype(jnp.int32) + offs).reshape(b * n * k // 128, 128)
    out = _pooled_gather(x_flat, gidx, n_rows=b * n, k=k, c=c)
    return out.reshape(b, n, c)


# speedup vs baseline: 1.0131x; 1.0131x over previous
import functools

import jax
import jax.numpy as jnp
from jax import lax
from jax.experimental import pallas as pl
from jax.experimental.pallas import tpu as pltpu
from jax.experimental.pallas import tpu_sc as plsc

NC = 2
NS = 16
L = 16


@functools.partial(jax.jit, static_argnames=("n_rows", "c"))
def _pack_bf16(x2d, *, n_rows, c):
    nw = NC * NS
    rpw = n_rows // nw
    ng = c // (2 * L)

    mesh = plsc.VectorSubcoreMesh(
        core_axis_name="c", subcore_axis_name="s", num_cores=NC,
        num_subcores=NS)

    @functools.partial(
        pl.kernel,
        mesh=mesh,
        compiler_params=pltpu.CompilerParams(use_tc_tiling_on_sc=False),
        out_type=jax.ShapeDtypeStruct((n_rows, c // 2), jnp.int32),
        scratch_types=[
            pltpu.VMEM((rpw, c), jnp.float32),
            pltpu.VMEM((rpw, c // 2), jnp.int32),
        ],
    )
    def run(x_hbm, out_hbm, inb, outb):
        wid = lax.axis_index("s") * NC + lax.axis_index("c")
        base = wid * rpw
        pltpu.sync_copy(x_hbm.at[pl.ds(base, rpw)], inb)

        one = jnp.full((L,), 1, jnp.int32)
        half = jnp.full((L,), 0x7FFF, jnp.int32)
        himask = jnp.full((L,), -65536, jnp.int32)

        def rnd(v):
            u = lax.bitcast_convert_type(v, jnp.int32)
            return u + half + (lax.shift_right_logical(u, 16) & one)

        @pl.loop(0, rpw, unroll=4)
        def _rows(rr):
            for g in range(ng):
                ra = rnd(inb[rr, pl.ds(g * L, L)])
                rb = rnd(inb[rr, pl.ds(c // 2 + g * L, L)])
                outb[rr, pl.ds(g * L, L)] = (
                    lax.shift_right_logical(ra, 16) | (rb & himask))

        pltpu.sync_copy(outb, out_hbm.at[pl.ds(base, rpw)])

    return run(x2d)


@functools.partial(jax.jit, static_argnames=("n_rows", "k", "c"))
def _pooled_gather(x_flat, gidx, *, n_rows, k, c):
    nw = NC * NS
    rpw = n_rows // nw
    gch = 128 // k
    nch = rpw // gch
    cv = c // L

    mesh = plsc.VectorSubcoreMesh(
        core_axis_name="c", subcore_axis_name="s", num_cores=NC,
        num_subcores=NS)

    ng = c // (2 * L)

    @functools.partial(
        pl.kernel,
        mesh=mesh,
        compiler_params=pltpu.CompilerParams(use_tc_tiling_on_sc=False),
        out_type=jax.ShapeDtypeStruct((n_rows * c,), jnp.float32),
        scratch_types=[
            pltpu.VMEM((rpw * k // 128, 128), jnp.int32),
            pltpu.VMEM((gch * k, c // 2), jnp.int32),
            pltpu.VMEM((gch * k, c // 2), jnp.int32),
            pltpu.VMEM((gch * k, c // 2), jnp.int32),
            pltpu.VMEM((gch * k, c // 2), jnp.int32),
            pltpu.VMEM((rpw * c,), jnp.float32),
            pltpu.SemaphoreType.DMA,
            pltpu.SemaphoreType.DMA,
            pltpu.SemaphoreType.DMA,
            pltpu.SemaphoreType.DMA,
        ],
    )
    def run(x_hbm, gidx_hbm, out_hbm, idx_v, gbuf0, gbuf1, gbuf2, gbuf3,
            obuf, sem0, sem1, sem2, sem3):
        wid = lax.axis_index("s") * NC + lax.axis_index("c")
        base = wid * rpw
        pltpu.sync_copy(
            gidx_hbm.at[pl.ds(wid * (rpw * k // 128), rpw * k // 128)], idx_v)

        himask = jnp.full((L,), -65536, jnp.int32)

        def issue(i, gbuf, sem):
            pltpu.async_copy(x_hbm.at[idx_v.at[i]], gbuf, sem)

        def drain(gbuf, sem):
            pltpu.make_async_copy(
                x_hbm.at[pl.ds(0, gch * k)], gbuf, sem).wait()

        def accumulate(i, gbuf):
            def acc_body(j, carry):
                new = list(carry)
                for r in range(gch):
                    for g in range(ng):
                        w = gbuf[r * k + j, pl.ds(g * L, L)]
                        ev = lax.bitcast_convert_type(
                            lax.shift_left(w, 16), jnp.float32)
                        od = lax.bitcast_convert_type(w & himask, jnp.float32)
                        new[(r * ng + g) * 2] += ev
                        new[(r * ng + g) * 2 + 1] += od
                return tuple(new)
            acc = lax.fori_loop(
                0, k, acc_body,
                tuple(jnp.zeros((L,), jnp.float32) for _ in range(gch * cv)),
                unroll=1)
            for r in range(gch):
                row_off = (i * gch + r) * c
                for g in range(ng):
                    obuf[pl.ds(row_off + g * L, L)] = (
                        acc[(r * ng + g) * 2] * (1.0 / k))
                    obuf[pl.ds(row_off + c // 2 + g * L, L)] = (
                        acc[(r * ng + g) * 2 + 1] * (1.0 / k))

        bufs = (gbuf0, gbuf1, gbuf2, gbuf3)
        sems = (sem0, sem1, sem2, sem3)
        nbuf = 4
        for b in range(nbuf - 1):
            issue(b, bufs[b], sems[b])

        @pl.loop(0, nch, step=nbuf)
        def _chunks(i):
            for b in range(nbuf):
                nxt = i + b + nbuf - 1

                @pl.when(nxt < nch)
                def _(b=b, nxt=nxt):
                    issue(nxt, bufs[(b + nbuf - 1) % nbuf],
                          sems[(b + nbuf - 1) % nbuf])

                drain(bufs[b], sems[b])
                accumulate(i + b, bufs[b])

        pltpu.sync_copy(obuf, out_hbm.at[pl.ds(base * c, rpw * c)])

    return run(x_flat, gidx)


def kernel(x, idx):
    b, n, c = x.shape
    k = idx.shape[-1]
    x_flat = _pack_bf16(x.reshape(b * n, c), n_rows=b * n, c=c)
    offs = (jnp.arange(b, dtype=jnp.int32) * n)[:, None, None]
    gidx = (idx.astype(jnp.int32) + offs).reshape(b * n * k // 128, 128)
    out = _pooled_gather(x_flat, gidx, n_rows=b * n, k=k, c=c)
    return out.reshape(b, n, c)
